# Initial kernel scaffold; baseline (speedup 1.0000x reference)
import jax
import jax.numpy as jnp
from jax.experimental import pallas as pl


def kernel(x, edge_index, W1_rel, b1, W1_root, W2_rel, b2, W2_root):
    raise NotImplementedError("write your pallas kernel here")


# speedup vs baseline: 10.2115x; 10.2115x over previous
import functools

import jax
import jax.numpy as jnp
from jax import lax
from jax.experimental import pallas as pl
from jax.experimental.pallas import tpu as pltpu
from jax.experimental.pallas import tpu_sc as plsc

_N = 10000
_D = 128
_H = 16
_E = 320000
_NC = 2
_NS = 16
_NW = _NC * _NS
_L = 16
_CHUNK = 128
_NCHUNK = 80
_EPW = _CHUNK * _NCHUNK
_EPAD = _EPW * _NW
_NACC = 10240
_RPT = _NACC // _NS
_NPAD = 10008

_f32 = jnp.float32


def _make_segsum():
    mesh = plsc.VectorSubcoreMesh(
        core_axis_name="c", subcore_axis_name="s",
        num_cores=_NC, num_subcores=_NS)

    @functools.partial(
        pl.kernel,
        out_type=jax.ShapeDtypeStruct((_NC, _NACC, _L), _f32),
        mesh=mesh,
        scratch_types=[
            pltpu.VMEM((_NCHUNK, _CHUNK), jnp.int32),
            pltpu.VMEM((_NCHUNK, _CHUNK), jnp.int32),
            pltpu.VMEM((_CHUNK, _L), _f32),
            pltpu.VMEM_SHARED((_NACC, _L), _f32),
            pltpu.SemaphoreType.DMA,
        ],
        compiler_params=pltpu.CompilerParams(use_tc_tiling_on_sc=False),
    )
    def segsum(p_hbm, src_hbm, dst_hbm, zeros_hbm, out_hbm,
               src_v, dst_v, rows_v, acc_sh, sem):
        cid = lax.axis_index("c")
        sid = lax.axis_index("s")
        wid = cid * _NS + sid
        row0 = sid * _RPT
        pltpu.sync_copy(zeros_hbm.at[pl.ds(row0, _RPT)],
                        acc_sh.at[pl.ds(row0, _RPT)])
        pltpu.sync_copy(src_hbm.at[wid], src_v)
        pltpu.sync_copy(dst_hbm.at[wid], dst_v)
        plsc.subcore_barrier()

        def body(j, carry):
            pltpu.async_copy(p_hbm.at[src_v.at[j]], rows_v, sem).wait()
            pltpu.sync_copy(rows_v, acc_sh.at[dst_v.at[j]], add=True)
            return carry

        lax.fori_loop(0, _NCHUNK, body, 0)
        plsc.subcore_barrier()
        pltpu.sync_copy(acc_sh.at[pl.ds(row0, _RPT)],
                        out_hbm.at[cid, pl.ds(row0, _RPT)])

    return segsum


_segsum = _make_segsum()


def _tc1_body(x_ref, wrel_ref, wroot_ref, b_ref, p_ref, r_ref):
    x = x_ref[...]
    p_ref[pl.ds(0, _N), :] = jnp.dot(x, wrel_ref[...],
                                     preferred_element_type=_f32)
    p_ref[pl.ds(_N, _NPAD - _N), :] = jnp.zeros((_NPAD - _N, _H), _f32)
    r_ref[...] = jnp.dot(x, wroot_ref[...],
                         preferred_element_type=_f32) + b_ref[...]


def _tc2_body(part_ref, r1_ref, wrel_ref, wroot_ref, b_ref, p2_ref, r2_ref):
    agg = part_ref[0, 0:_N, :] + part_ref[1, 0:_N, :]
    h = jnp.maximum(agg + r1_ref[...], 0.0)
    p2_ref[pl.ds(0, _N), :] = jnp.dot(h, wrel_ref[...],
                                      preferred_element_type=_f32)
    p2_ref[pl.ds(_N, _NPAD - _N), :] = jnp.zeros((_NPAD - _N, _H), _f32)
    r2_ref[...] = jnp.dot(h, wroot_ref[...],
                          preferred_element_type=_f32) + b_ref[...]


def _tc3_body(part_ref, r2_ref, o_ref):
    z = part_ref[0, 0:_N, :] + part_ref[1, 0:_N, :] + r2_ref[...]
    m = jnp.max(z, axis=1, keepdims=True)
    zm = z - m
    o_ref[...] = zm - jnp.log(jnp.sum(jnp.exp(zm), axis=1, keepdims=True))


_tc1 = pl.pallas_call(
    _tc1_body,
    out_shape=(jax.ShapeDtypeStruct((_NPAD, _H), _f32),
               jax.ShapeDtypeStruct((_N, _H), _f32)))

_tc2 = pl.pallas_call(
    _tc2_body,
    out_shape=(jax.ShapeDtypeStruct((_NPAD, _H), _f32),
               jax.ShapeDtypeStruct((_N, _H), _f32)))

_tc3 = pl.pallas_call(
    _tc3_body,
    out_shape=jax.ShapeDtypeStruct((_N, _H), _f32))


def kernel(x, edge_index, W1_rel, b1, W1_root, W2_rel, b2, W2_root):
    src = edge_index[0]
    dst = edge_index[1]
    pad = _EPAD - _E
    src_p = jnp.concatenate(
        [src, jnp.full((pad,), _N, jnp.int32)]).reshape(_NW, _NCHUNK, _CHUNK)
    dst_p = jnp.concatenate(
        [dst, jnp.full((pad,), _N, jnp.int32)]).reshape(_NW, _NCHUNK, _CHUNK)
    zeros = jnp.zeros((_NACC, _L), _f32)

    p1, r1 = _tc1(x, W1_rel, W1_root, b1.reshape(1, _H))
    part1 = _segsum(p1, src_p, dst_p, zeros)
    p2, r2 = _tc2(part1, r1, W2_rel, W2_root, b2.reshape(1, _H))
    part2 = _segsum(p2, src_p, dst_p, zeros)
    return _tc3(part2, r2)


# speedup vs baseline: 13.8266x; 1.3540x over previous
import functools

import jax
import jax.numpy as jnp
from jax import lax
from jax.experimental import pallas as pl
from jax.experimental.pallas import tpu as pltpu
from jax.experimental.pallas import tpu_sc as plsc

_N = 10000
_D = 128
_H = 16
_E = 320000
_NC = 2
_NS = 16
_NW = _NC * _NS
_L = 16
_CHUNK = 128
_NCHUNK = 80
_EPW = _CHUNK * _NCHUNK
_EPAD = _EPW * _NW
_NACC = 10240
_RPT = _NACC // _NS
_NPAD = 10008

_f32 = jnp.float32


def _make_segsum():
    mesh = plsc.VectorSubcoreMesh(
        core_axis_name="c", subcore_axis_name="s",
        num_cores=_NC, num_subcores=_NS)

    @functools.partial(
        pl.kernel,
        out_type=jax.ShapeDtypeStruct((_NC, _NACC, _L), _f32),
        mesh=mesh,
        scratch_types=[
            pltpu.VMEM((_NCHUNK, _CHUNK), jnp.int32),
            pltpu.VMEM((_NCHUNK, _CHUNK), jnp.int32),
            pltpu.VMEM((_CHUNK, _L), _f32),
            pltpu.VMEM((_CHUNK, _L), _f32),
            pltpu.VMEM_SHARED((_NACC, _L), _f32),
            pltpu.SemaphoreType.DMA,
            pltpu.SemaphoreType.DMA,
        ],
        compiler_params=pltpu.CompilerParams(use_tc_tiling_on_sc=False),
    )
    def segsum(p_hbm, src_hbm, dst_hbm, zeros_hbm, out_hbm,
               src_v, dst_v, rows0_v, rows1_v, acc_sh, sem0, sem1):
        cid = lax.axis_index("c")
        sid = lax.axis_index("s")
        wid = cid * _NS + sid
        row0 = sid * _RPT
        pltpu.sync_copy(zeros_hbm.at[pl.ds(row0, _RPT)],
                        acc_sh.at[pl.ds(row0, _RPT)])
        pltpu.sync_copy(src_hbm.at[wid], src_v)
        pltpu.sync_copy(dst_hbm.at[wid], dst_v)
        plsc.subcore_barrier()

        pltpu.async_copy(p_hbm.at[src_v.at[0]], rows0_v, sem0)

        def body(i, carry):
            c0 = 2 * i
            pltpu.async_copy(p_hbm.at[src_v.at[c0 + 1]], rows1_v, sem1)
            pltpu.make_async_copy(p_hbm.at[src_v.at[c0]], rows0_v, sem0).wait()
            pltpu.sync_copy(rows0_v, acc_sh.at[dst_v.at[c0]], add=True)

            @pl.when(i < _NCHUNK // 2 - 1)
            def _():
                pltpu.async_copy(p_hbm.at[src_v.at[c0 + 2]], rows0_v, sem0)

            pltpu.make_async_copy(p_hbm.at[src_v.at[c0 + 1]], rows1_v,
                                  sem1).wait()
            pltpu.sync_copy(rows1_v, acc_sh.at[dst_v.at[c0 + 1]], add=True)
            return carry

        lax.fori_loop(0, _NCHUNK // 2, body, 0)
        plsc.subcore_barrier()
        pltpu.sync_copy(acc_sh.at[pl.ds(row0, _RPT)],
                        out_hbm.at[cid, pl.ds(row0, _RPT)])

    return segsum


_segsum = _make_segsum()


def _tc1_body(x_ref, wrel_ref, wroot_ref, b_ref, p_ref, r_ref):
    x = x_ref[...]
    p_ref[pl.ds(0, _N), :] = jnp.dot(x, wrel_ref[...],
                                     preferred_element_type=_f32)
    p_ref[pl.ds(_N, _NPAD - _N), :] = jnp.zeros((_NPAD - _N, _H), _f32)
    r_ref[...] = jnp.dot(x, wroot_ref[...],
                         preferred_element_type=_f32) + b_ref[...]


def _tc2_body(part_ref, r1_ref, wrel_ref, wroot_ref, b_ref, p2_ref, r2_ref):
    agg = part_ref[0, 0:_N, :] + part_ref[1, 0:_N, :]
    h = jnp.maximum(agg + r1_ref[...], 0.0)
    p2_ref[pl.ds(0, _N), :] = jnp.dot(h, wrel_ref[...],
                                      preferred_element_type=_f32)
    p2_ref[pl.ds(_N, _NPAD - _N), :] = jnp.zeros((_NPAD - _N, _H), _f32)
    r2_ref[...] = jnp.dot(h, wroot_ref[...],
                          preferred_element_type=_f32) + b_ref[...]


def _tc3_body(part_ref, r2_ref, o_ref):
    z = part_ref[0, 0:_N, :] + part_ref[1, 0:_N, :] + r2_ref[...]
    m = jnp.max(z, axis=1, keepdims=True)
    zm = z - m
    o_ref[...] = zm - jnp.log(jnp.sum(jnp.exp(zm), axis=1, keepdims=True))


_tc1 = pl.pallas_call(
    _tc1_body,
    out_shape=(jax.ShapeDtypeStruct((_NPAD, _H), _f32),
               jax.ShapeDtypeStruct((_N, _H), _f32)))

_tc2 = pl.pallas_call(
    _tc2_body,
    out_shape=(jax.ShapeDtypeStruct((_NPAD, _H), _f32),
               jax.ShapeDtypeStruct((_N, _H), _f32)))

_tc3 = pl.pallas_call(
    _tc3_body,
    out_shape=jax.ShapeDtypeStruct((_N, _H), _f32))


def kernel(x, edge_index, W1_rel, b1, W1_root, W2_rel, b2, W2_root):
    src = edge_index[0]
    dst = edge_index[1]
    pad = _EPAD - _E
    src_p = jnp.concatenate(
        [src, jnp.full((pad,), _N, jnp.int32)]).reshape(_NW, _NCHUNK, _CHUNK)
    dst_p = jnp.concatenate(
        [dst, jnp.full((pad,), _N, jnp.int32)]).reshape(_NW, _NCHUNK, _CHUNK)
    zeros = jnp.zeros((_NACC, _L), _f32)

    p1, r1 = _tc1(x, W1_rel, W1_root, b1.reshape(1, _H))
    part1 = _segsum(p1, src_p, dst_p, zeros)
    p2, r2 = _tc2(part1, r1, W2_rel, W2_root, b2.reshape(1, _H))
    part2 = _segsum(p2, src_p, dst_p, zeros)
    return _tc3(part2, r2)


# speedup vs baseline: 14.1383x; 1.0225x over previous
import functools

import jax
import jax.numpy as jnp
from jax import lax
from jax.experimental import pallas as pl
from jax.experimental.pallas import tpu as pltpu
from jax.experimental.pallas import tpu_sc as plsc

_N = 10000
_D = 128
_H = 16
_E = 320000
_NC = 2
_NS = 16
_NW = _NC * _NS
_L = 16
_CHUNK = 128
_NCHUNK = 80
_EPW = _CHUNK * _NCHUNK
_EPAD = _EPW * _NW
_NACC = 10240
_RPT = _NACC // _NS
_NPAD = 10008

_f32 = jnp.float32


def _make_segsum():
    mesh = plsc.VectorSubcoreMesh(
        core_axis_name="c", subcore_axis_name="s",
        num_cores=_NC, num_subcores=_NS)

    @functools.partial(
        pl.kernel,
        out_type=jax.ShapeDtypeStruct((_NC, _NACC, _L), _f32),
        mesh=mesh,
        scratch_types=[
            pltpu.VMEM((_NCHUNK, _CHUNK), jnp.int32),
            pltpu.VMEM((_NCHUNK, _CHUNK), jnp.int32),
            pltpu.VMEM((4, _CHUNK, _L), _f32),
            pltpu.VMEM_SHARED((_NACC, _L), _f32),
            pltpu.SemaphoreType.DMA,
            pltpu.SemaphoreType.DMA,
            pltpu.SemaphoreType.DMA,
            pltpu.SemaphoreType.DMA,
            pltpu.SemaphoreType.DMA,
            pltpu.SemaphoreType.DMA,
            pltpu.SemaphoreType.DMA,
            pltpu.SemaphoreType.DMA,
        ],
        compiler_params=pltpu.CompilerParams(use_tc_tiling_on_sc=False),
    )
    def segsum(p_hbm, src_hbm, dst_hbm, zeros_hbm, out_hbm,
               src_v, dst_v, ring_v, acc_sh,
               g0, g1, g2, g3, s0, s1, s2, s3):
        gsem = (g0, g1, g2, g3)
        ssem = (s0, s1, s2, s3)
        cid = lax.axis_index("c")
        sid = lax.axis_index("s")
        wid = cid * _NS + sid
        row0 = sid * _RPT
        pltpu.sync_copy(zeros_hbm.at[pl.ds(row0, _RPT)],
                        acc_sh.at[pl.ds(row0, _RPT)])
        pltpu.sync_copy(src_hbm.at[wid], src_v)
        pltpu.sync_copy(dst_hbm.at[wid], dst_v)
        plsc.subcore_barrier()

        def _gather(c, b):
            pltpu.async_copy(p_hbm.at[src_v.at[c]], ring_v.at[b], gsem[b])

        def _gwait(c, b):
            pltpu.make_async_copy(p_hbm.at[src_v.at[c]], ring_v.at[b],
                                  gsem[b]).wait()

        def _scat(c, b):
            pltpu.async_copy(ring_v.at[b], acc_sh.at[dst_v.at[c]], ssem[b],
                             add=True)

        def _swait(c, b):
            pltpu.make_async_copy(ring_v.at[b], acc_sh.at[dst_v.at[c]],
                                  ssem[b]).wait()

        _gather(0, 0)
        _gather(1, 1)
        _ngrp = _NCHUNK // 4

        def body(g, carry):
            c0 = 4 * g
            for b in range(4):
                c = c0 + b
                pb = (b + 2) % 4
                if b < 2:
                    @pl.when(g >= 1)
                    def _(c=c, pb=pb):
                        _swait(c - 2, pb)
                    _gather(c + 2, pb)
                else:
                    _swait(c - 2, pb)

                    @pl.when(g < _ngrp - 1)
                    def _(c=c, pb=pb):
                        _gather(c + 2, pb)
                _gwait(c, b)
                _scat(c, b)
            return carry

        lax.fori_loop(0, _ngrp, body, 0)
        _swait(_NCHUNK - 2, 2)
        _swait(_NCHUNK - 1, 3)
        plsc.subcore_barrier()
        pltpu.sync_copy(acc_sh.at[pl.ds(row0, _RPT)],
                        out_hbm.at[cid, pl.ds(row0, _RPT)])

    return segsum


_segsum = _make_segsum()


def _tc1_body(x_ref, wrel_ref, wroot_ref, b_ref, p_ref, r_ref):
    x = x_ref[...]
    p_ref[pl.ds(0, _N), :] = jnp.dot(x, wrel_ref[...],
                                     preferred_element_type=_f32)
    p_ref[pl.ds(_N, _NPAD - _N), :] = jnp.zeros((_NPAD - _N, _H), _f32)
    r_ref[...] = jnp.dot(x, wroot_ref[...],
                         preferred_element_type=_f32) + b_ref[...]


def _tc2_body(part_ref, r1_ref, wrel_ref, wroot_ref, b_ref, p2_ref, r2_ref):
    agg = part_ref[0, 0:_N, :] + part_ref[1, 0:_N, :]
    h = jnp.maximum(agg + r1_ref[...], 0.0)
    p2_ref[pl.ds(0, _N), :] = jnp.dot(h, wrel_ref[...],
                                      preferred_element_type=_f32)
    p2_ref[pl.ds(_N, _NPAD - _N), :] = jnp.zeros((_NPAD - _N, _H), _f32)
    r2_ref[...] = jnp.dot(h, wroot_ref[...],
                          preferred_element_type=_f32) + b_ref[...]


def _tc3_body(part_ref, r2_ref, o_ref):
    z = part_ref[0, 0:_N, :] + part_ref[1, 0:_N, :] + r2_ref[...]
    m = jnp.max(z, axis=1, keepdims=True)
    zm = z - m
    o_ref[...] = zm - jnp.log(jnp.sum(jnp.exp(zm), axis=1, keepdims=True))


_tc1 = pl.pallas_call(
    _tc1_body,
    out_shape=(jax.ShapeDtypeStruct((_NPAD, _H), _f32),
               jax.ShapeDtypeStruct((_N, _H), _f32)))

_tc2 = pl.pallas_call(
    _tc2_body,
    out_shape=(jax.ShapeDtypeStruct((_NPAD, _H), _f32),
               jax.ShapeDtypeStruct((_N, _H), _f32)))

_tc3 = pl.pallas_call(
    _tc3_body,
    out_shape=jax.ShapeDtypeStruct((_N, _H), _f32))


def kernel(x, edge_index, W1_rel, b1, W1_root, W2_rel, b2, W2_root):
    src = edge_index[0]
    dst = edge_index[1]
    pad = _EPAD - _E
    src_p = jnp.concatenate(
        [src, jnp.full((pad,), _N, jnp.int32)]).reshape(_NW, _NCHUNK, _CHUNK)
    dst_p = jnp.concatenate(
        [dst, jnp.full((pad,), _N, jnp.int32)]).reshape(_NW, _NCHUNK, _CHUNK)
    zeros = jnp.zeros((_NACC, _L), _f32)

    p1, r1 = _tc1(x, W1_rel, W1_root, b1.reshape(1, _H))
    part1 = _segsum(p1, src_p, dst_p, zeros)
    p2, r2 = _tc2(part1, r1, W2_rel, W2_root, b2.reshape(1, _H))
    part2 = _segsum(p2, src_p, dst_p, zeros)
    return _tc3(part2, r2)


# speedup vs baseline: 20.6694x; 1.4620x over previous
import functools

import jax
import jax.numpy as jnp
from jax import lax
from jax.experimental import pallas as pl
from jax.experimental.pallas import tpu as pltpu
from jax.experimental.pallas import tpu_sc as plsc

_N = 10000
_D = 128
_H = 16
_E = 320000
_NC = 2
_NS = 16
_NW = _NC * _NS
_L = 16
_CHUNK = 125
_NCHUNK = 80
_NACC = 10240
_RPT = _NACC // _NS

_f32 = jnp.float32


def _make_segsum():
    mesh = plsc.VectorSubcoreMesh(
        core_axis_name="c", subcore_axis_name="s",
        num_cores=_NC, num_subcores=_NS)

    @functools.partial(
        pl.kernel,
        out_type=jax.ShapeDtypeStruct((_NC, _NACC, _L), _f32),
        mesh=mesh,
        scratch_types=[
            pltpu.VMEM((_NCHUNK, _CHUNK), jnp.int32),
            pltpu.VMEM((_NCHUNK, _CHUNK), jnp.int32),
            pltpu.VMEM((4, _CHUNK, _L), _f32),
            pltpu.VMEM((_RPT, _L), _f32),
            pltpu.VMEM_SHARED((_NACC, _L), _f32),
            pltpu.SemaphoreType.DMA,
            pltpu.SemaphoreType.DMA,
            pltpu.SemaphoreType.DMA,
            pltpu.SemaphoreType.DMA,
            pltpu.SemaphoreType.DMA,
            pltpu.SemaphoreType.DMA,
            pltpu.SemaphoreType.DMA,
            pltpu.SemaphoreType.DMA,
        ],
        compiler_params=pltpu.CompilerParams(use_tc_tiling_on_sc=False),
    )
    def segsum(p_hbm, ei_hbm, out_hbm,
               src_v, dst_v, ring_v, zero_v, acc_sh,
               g0, g1, g2, g3, s0, s1, s2, s3):
        gsem = (g0, g1, g2, g3)
        ssem = (s0, s1, s2, s3)
        cid = lax.axis_index("c")
        sid = lax.axis_index("s")
        wid = cid * _NS + sid
        row0 = sid * _RPT

        def zbody(i, carry):
            zero_v[i] = jnp.zeros((_L,), _f32)
            return carry

        lax.fori_loop(0, _RPT, zbody, 0)
        pltpu.sync_copy(zero_v, acc_sh.at[pl.ds(row0, _RPT)])
        pltpu.sync_copy(ei_hbm.at[0, wid], src_v)
        pltpu.sync_copy(ei_hbm.at[1, wid], dst_v)
        plsc.subcore_barrier()

        def _gather(c, b):
            pltpu.async_copy(p_hbm.at[src_v.at[c]], ring_v.at[b], gsem[b])

        def _gwait(c, b):
            pltpu.make_async_copy(p_hbm.at[src_v.at[c]], ring_v.at[b],
                                  gsem[b]).wait()

        def _scat(c, b):
            pltpu.async_copy(ring_v.at[b], acc_sh.at[dst_v.at[c]], ssem[b],
                             add=True)

        def _swait(c, b):
            pltpu.make_async_copy(ring_v.at[b], acc_sh.at[dst_v.at[c]],
                                  ssem[b]).wait()

        _gather(0, 0)
        _gather(1, 1)
        _ngrp = _NCHUNK // 4

        def body(g, carry):
            c0 = 4 * g
            for b in range(4):
                c = c0 + b
                pb = (b + 2) % 4
                if b < 2:
                    @pl.when(g >= 1)
                    def _(c=c, pb=pb):
                        _swait(c - 2, pb)
                    _gather(c + 2, pb)
                else:
                    _swait(c - 2, pb)

                    @pl.when(g < _ngrp - 1)
                    def _(c=c, pb=pb):
                        _gather(c + 2, pb)
                _gwait(c, b)
                _scat(c, b)
            return carry

        lax.fori_loop(0, _ngrp, body, 0)
        _swait(_NCHUNK - 2, 2)
        _swait(_NCHUNK - 1, 3)
        plsc.subcore_barrier()
        pltpu.sync_copy(acc_sh.at[pl.ds(row0, _RPT)],
                        out_hbm.at[cid, pl.ds(row0, _RPT)])

    return segsum


_segsum = _make_segsum()


def _tc1_body(x_ref, wrel_ref, wroot_ref, b_ref, p_ref, r_ref):
    x = x_ref[...]
    p_ref[...] = jnp.dot(x, wrel_ref[...], preferred_element_type=_f32)
    r_ref[...] = jnp.dot(x, wroot_ref[...],
                         preferred_element_type=_f32) + b_ref[...]


def _tc2_body(part_ref, r1_ref, wrel_ref, wroot_ref, b_ref, p2_ref, r2_ref):
    agg = part_ref[0, 0:_N, :] + part_ref[1, 0:_N, :]
    h = jnp.maximum(agg + r1_ref[...], 0.0)
    p2_ref[...] = jnp.dot(h, wrel_ref[...], preferred_element_type=_f32)
    r2_ref[...] = jnp.dot(h, wroot_ref[...],
                          preferred_element_type=_f32) + b_ref[...]


def _tc3_body(part_ref, r2_ref, o_ref):
    z = part_ref[0, 0:_N, :] + part_ref[1, 0:_N, :] + r2_ref[...]
    m = jnp.max(z, axis=1, keepdims=True)
    zm = z - m
    o_ref[...] = zm - jnp.log(jnp.sum(jnp.exp(zm), axis=1, keepdims=True))


_tc1 = pl.pallas_call(
    _tc1_body,
    out_shape=(jax.ShapeDtypeStruct((_N, _H), _f32),
               jax.ShapeDtypeStruct((_N, _H), _f32)))

_tc2 = pl.pallas_call(
    _tc2_body,
    out_shape=(jax.ShapeDtypeStruct((_N, _H), _f32),
               jax.ShapeDtypeStruct((_N, _H), _f32)))

_tc3 = pl.pallas_call(
    _tc3_body,
    out_shape=jax.ShapeDtypeStruct((_N, _H), _f32))


def kernel(x, edge_index, W1_rel, b1, W1_root, W2_rel, b2, W2_root):
    ei = edge_index.reshape(2, _NW, _NCHUNK, _CHUNK)

    p1, r1 = _tc1(x, W1_rel, W1_root, b1.reshape(1, _H))
    part1 = _segsum(p1, ei)
    p2, r2 = _tc2(part1, r1, W2_rel, W2_root, b2.reshape(1, _H))
    part2 = _segsum(p2, ei)
    return _tc3(part2, r2)


# speedup vs baseline: 24.7746x; 1.1986x over previous
import functools

import jax
import jax.numpy as jnp
from jax import lax
from jax.experimental import pallas as pl
from jax.experimental.pallas import tpu as pltpu
from jax.experimental.pallas import tpu_sc as plsc

_N = 10000
_D = 128
_H = 16
_E = 320000
_NC = 2
_NS = 16
_NW = _NC * _NS
_L = 16
_CHUNK = 125
_NCHUNK = 80
_NACC = 10240
_RPT = _NACC // _NS

_f32 = jnp.float32


def _make_segsum():
    mesh = plsc.VectorSubcoreMesh(
        core_axis_name="c", subcore_axis_name="s",
        num_cores=_NC, num_subcores=_NS)

    @functools.partial(
        pl.kernel,
        out_type=jax.ShapeDtypeStruct((_NC, _NACC, _L), _f32),
        mesh=mesh,
        scratch_types=[
            pltpu.VMEM((_NCHUNK, _CHUNK), jnp.int32),
            pltpu.VMEM((_NCHUNK, _CHUNK), jnp.int32),
            pltpu.VMEM((4, _CHUNK, _L), _f32),
            pltpu.VMEM((_RPT, _L), _f32),
            pltpu.VMEM_SHARED((_NACC, _L), _f32),
            pltpu.SemaphoreType.DMA,
            pltpu.SemaphoreType.DMA,
            pltpu.SemaphoreType.DMA,
            pltpu.SemaphoreType.DMA,
            pltpu.SemaphoreType.DMA,
            pltpu.SemaphoreType.DMA,
            pltpu.SemaphoreType.DMA,
            pltpu.SemaphoreType.DMA,
        ],
        compiler_params=pltpu.CompilerParams(use_tc_tiling_on_sc=False),
    )
    def segsum(p_hbm, ei_hbm, out_hbm,
               src_v, dst_v, ring_v, zero_v, acc_sh,
               g0, g1, g2, g3, s0, s1, s2, s3):
        gsem = (g0, g1, g2, g3)
        ssem = (s0, s1, s2, s3)
        cid = lax.axis_index("c")
        sid = lax.axis_index("s")
        wid = cid * _NS + sid
        row0 = sid * _RPT

        def zbody(i, carry):
            zero_v[i] = jnp.zeros((_L,), _f32)
            return carry

        lax.fori_loop(0, _RPT, zbody, 0)
        pltpu.sync_copy(zero_v, acc_sh.at[pl.ds(row0, _RPT)])
        pltpu.sync_copy(ei_hbm.at[0, wid], src_v)
        pltpu.sync_copy(ei_hbm.at[1, wid], dst_v)
        plsc.subcore_barrier()

        def _gather(c, b):
            pltpu.async_copy(p_hbm.at[src_v.at[c]], ring_v.at[b], gsem[b])

        def _gwait(c, b):
            pltpu.make_async_copy(p_hbm.at[src_v.at[c]], ring_v.at[b],
                                  gsem[b]).wait()

        def _scat(c, b):
            pltpu.async_copy(ring_v.at[b], acc_sh.at[dst_v.at[c]], ssem[b],
                             add=True)

        def _swait(c, b):
            pltpu.make_async_copy(ring_v.at[b], acc_sh.at[dst_v.at[c]],
                                  ssem[b]).wait()

        _gather(0, 0)
        _gather(1, 1)
        _ngrp = _NCHUNK // 4

        def body(g, carry):
            c0 = 4 * g
            for b in range(4):
                c = c0 + b
                pb = (b + 2) % 4
                if b < 2:
                    @pl.when(g >= 1)
                    def _(c=c, pb=pb):
                        _swait(c - 2, pb)
                    _gather(c + 2, pb)
                else:
                    _swait(c - 2, pb)

                    @pl.when(g < _ngrp - 1)
                    def _(c=c, pb=pb):
                        _gather(c + 2, pb)
                _gwait(c, b)
                _scat(c, b)
            return carry

        lax.fori_loop(0, _ngrp, body, 0)
        _swait(_NCHUNK - 2, 2)
        _swait(_NCHUNK - 1, 3)
        plsc.subcore_barrier()
        pltpu.sync_copy(acc_sh.at[pl.ds(row0, _RPT)],
                        out_hbm.at[cid, pl.ds(row0, _RPT)])

    return segsum


_segsum = _make_segsum()


_NP = _N // 8
_NACCP = _NACC // 8


def _tc1_body(x_ref, wrel_ref, wroot_ref, b_ref, p_ref, r_ref):
    x = x_ref[...]
    p_ref[...] = jnp.dot(x, wrel_ref[...], preferred_element_type=_f32)
    r_ref[...] = jnp.dot(x, wroot_ref[...],
                         preferred_element_type=_f32) + b_ref[...]


def _tc2_body(part_ref, r1_ref, wrel_ref, wroot_ref, b_ref, p2_ref, r2_ref):
    agg = part_ref[0, 0:_NP, :] + part_ref[1, 0:_NP, :]
    h = jnp.maximum(agg + r1_ref[...], 0.0)
    p2_ref[...] = jnp.dot(h, wrel_ref[...], preferred_element_type=_f32)
    r2_ref[...] = jnp.dot(h, wroot_ref[...],
                          preferred_element_type=_f32) + b_ref[...]


def _tc3_body(part_ref, r2_ref, o_ref):
    z = part_ref[0, 0:_NP, :] + part_ref[1, 0:_NP, :] + r2_ref[...]
    zg = jnp.reshape(z, (_NP, 8, _H))
    m = jnp.max(zg, axis=2, keepdims=True)
    zm = zg - m
    lse = jnp.log(jnp.sum(jnp.exp(zm), axis=2, keepdims=True))
    o_ref[...] = jnp.reshape(zm - lse, (_NP, 128))


_tc1 = pl.pallas_call(
    _tc1_body,
    out_shape=(jax.ShapeDtypeStruct((_NP, 128), _f32),
               jax.ShapeDtypeStruct((_NP, 128), _f32)))

_tc2 = pl.pallas_call(
    _tc2_body,
    out_shape=(jax.ShapeDtypeStruct((_NP, 128), _f32),
               jax.ShapeDtypeStruct((_NP, 128), _f32)))

_tc3 = pl.pallas_call(
    _tc3_body,
    out_shape=jax.ShapeDtypeStruct((_NP, 128), _f32))


def _blockdiag8(w):
    k, m = w.shape
    eye = jnp.eye(8, dtype=w.dtype)
    return (eye[:, None, :, None] * w[None, :, None, :]).reshape(8 * k, 8 * m)


def kernel(x, edge_index, W1_rel, b1, W1_root, W2_rel, b2, W2_root):
    ei = edge_index.reshape(2, _NW, _NCHUNK, _CHUNK)
    x8 = x.reshape(_NP, 8 * _D)

    p1p, r1p = _tc1(x8, _blockdiag8(W1_rel), _blockdiag8(W1_root),
                    jnp.tile(b1, 8).reshape(1, 128))
    part1 = _segsum(p1p.reshape(_N, _H), ei)
    p2p, r2p = _tc2(part1.reshape(_NC, _NACCP, 128), r1p,
                    _blockdiag8(W2_rel), _blockdiag8(W2_root),
                    jnp.tile(b2, 8).reshape(1, 128))
    part2 = _segsum(p2p.reshape(_N, _H), ei)
    outp = _tc3(part2.reshape(_NC, _NACCP, 128), r2p)
    return outp.reshape(_N, _H)


# speedup vs baseline: 25.1221x; 1.0140x over previous
import functools

import jax
import jax.numpy as jnp
from jax import lax
from jax.experimental import pallas as pl
from jax.experimental.pallas import tpu as pltpu
from jax.experimental.pallas import tpu_sc as plsc

_N = 10000
_D = 128
_H = 16
_E = 320000
_NC = 2
_NS = 16
_NW = _NC * _NS
_L = 16
_CHUNK = 125
_NCHUNK = 80
_NACC = 10240
_RPT = _NACC // _NS

_f32 = jnp.float32


def _make_segsum():
    mesh = plsc.VectorSubcoreMesh(
        core_axis_name="c", subcore_axis_name="s",
        num_cores=_NC, num_subcores=_NS)

    @functools.partial(
        pl.kernel,
        out_type=jax.ShapeDtypeStruct((_NC, _NACC, _L), _f32),
        mesh=mesh,
        scratch_types=[
            pltpu.VMEM((_NCHUNK, _CHUNK), jnp.int32),
            pltpu.VMEM((_NCHUNK, _CHUNK), jnp.int32),
            pltpu.VMEM((4, _CHUNK, _L), _f32),
            pltpu.VMEM((_RPT, _L), _f32),
            pltpu.VMEM_SHARED((_NACC, _L), _f32),
            pltpu.SemaphoreType.DMA,
            pltpu.SemaphoreType.DMA,
            pltpu.SemaphoreType.DMA,
            pltpu.SemaphoreType.DMA,
            pltpu.SemaphoreType.DMA,
            pltpu.SemaphoreType.DMA,
            pltpu.SemaphoreType.DMA,
            pltpu.SemaphoreType.DMA,
        ],
        compiler_params=pltpu.CompilerParams(use_tc_tiling_on_sc=False),
    )
    def segsum(p_hbm, ei_hbm, out_hbm,
               src_v, dst_v, ring_v, zero_v, acc_sh,
               g0, g1, g2, g3, s0, s1, s2, s3):
        gsem = (g0, g1, g2, g3)
        ssem = (s0, s1, s2, s3)
        cid = lax.axis_index("c")
        sid = lax.axis_index("s")
        wid = cid * _NS + sid
        row0 = sid * _RPT

        def zbody(i, carry):
            zero_v[i] = jnp.zeros((_L,), _f32)
            return carry

        lax.fori_loop(0, _RPT, zbody, 0)
        pltpu.sync_copy(zero_v, acc_sh.at[pl.ds(row0, _RPT)])
        pltpu.sync_copy(ei_hbm.at[0, wid], src_v)
        pltpu.sync_copy(ei_hbm.at[1, wid], dst_v)
        plsc.subcore_barrier()

        def _gather(c, b):
            pltpu.async_copy(p_hbm.at[src_v.at[c]], ring_v.at[b], gsem[b])

        def _gwait(c, b):
            pltpu.make_async_copy(p_hbm.at[src_v.at[c]], ring_v.at[b],
                                  gsem[b]).wait()

        def _scat(c, b):
            pltpu.async_copy(ring_v.at[b], acc_sh.at[dst_v.at[c]], ssem[b],
                             add=True)

        def _swait(c, b):
            pltpu.make_async_copy(ring_v.at[b], acc_sh.at[dst_v.at[c]],
                                  ssem[b]).wait()

        _gather(0, 0)
        _gather(1, 1)
        _ngrp = _NCHUNK // 4

        def body(g, carry):
            c0 = 4 * g
            for b in range(4):
                c = c0 + b
                pb = (b + 2) % 4
                if b < 2:
                    @pl.when(g >= 1)
                    def _(c=c, pb=pb):
                        _swait(c - 2, pb)
                    _gather(c + 2, pb)
                else:
                    _swait(c - 2, pb)

                    @pl.when(g < _ngrp - 1)
                    def _(c=c, pb=pb):
                        _gather(c + 2, pb)
                _gwait(c, b)
                _scat(c, b)
            return carry

        lax.fori_loop(0, _ngrp, body, 0)
        _swait(_NCHUNK - 2, 2)
        _swait(_NCHUNK - 1, 3)
        plsc.subcore_barrier()
        pltpu.sync_copy(acc_sh.at[pl.ds(row0, _RPT)],
                        out_hbm.at[cid, pl.ds(row0, _RPT)])

    return segsum


_segsum = _make_segsum()


_NP = _N // 8
_NACCP = _NACC // 8


def _tc1_body(x_ref, wrel_ref, wroot_ref, b_ref, p_ref, r_ref):
    x = x_ref[...]
    p_ref[...] = jnp.dot(x, wrel_ref[...], preferred_element_type=_f32)
    r_ref[...] = jnp.dot(x, wroot_ref[...],
                         preferred_element_type=_f32) + b_ref[...]


def _tc2_body(part_ref, r1_ref, wrel_ref, wroot_ref, b_ref, p2_ref, r2_ref):
    agg = part_ref[0, 0:_NP, :] + part_ref[1, 0:_NP, :]
    h = jnp.maximum(agg + r1_ref[...], 0.0)
    p2_ref[...] = jnp.dot(h, wrel_ref[...], preferred_element_type=_f32)
    r2_ref[...] = jnp.dot(h, wroot_ref[...],
                          preferred_element_type=_f32) + b_ref[...]


def _tc3_body(part_ref, r2_ref, gsum_ref, o_ref):
    z = part_ref[0, 0:_NP, :] + part_ref[1, 0:_NP, :] + r2_ref[...]
    zg = jnp.reshape(z, (_NP, 8, _H))
    m = jnp.max(zg, axis=2, keepdims=True)
    mp = jnp.reshape(jnp.broadcast_to(m, (_NP, 8, _H)), (_NP, 128))
    zm = z - mp
    e = jnp.exp(zm)
    s = jnp.dot(e, gsum_ref[...], preferred_element_type=_f32)
    o_ref[...] = zm - jnp.log(s)


_tc1 = pl.pallas_call(
    _tc1_body,
    out_shape=(jax.ShapeDtypeStruct((_NP, 128), _f32),
               jax.ShapeDtypeStruct((_NP, 128), _f32)))

_tc2 = pl.pallas_call(
    _tc2_body,
    out_shape=(jax.ShapeDtypeStruct((_NP, 128), _f32),
               jax.ShapeDtypeStruct((_NP, 128), _f32)))

_tc3 = pl.pallas_call(
    _tc3_body,
    out_shape=jax.ShapeDtypeStruct((_NP, 128), _f32))


def _blockdiag8(w):
    k, m = w.shape
    eye = jnp.eye(8, dtype=w.dtype)
    return (eye[:, None, :, None] * w[None, :, None, :]).reshape(8 * k, 8 * m)


def kernel(x, edge_index, W1_rel, b1, W1_root, W2_rel, b2, W2_root):
    ei = edge_index.reshape(2, _NW, _NCHUNK, _CHUNK)
    x8 = x.reshape(_NP, 8 * _D)

    p1p, r1p = _tc1(x8, _blockdiag8(W1_rel), _blockdiag8(W1_root),
                    jnp.tile(b1, 8).reshape(1, 128))
    part1 = _segsum(p1p.reshape(_N, _H), ei)
    p2p, r2p = _tc2(part1.reshape(_NC, _NACCP, 128), r1p,
                    _blockdiag8(W2_rel), _blockdiag8(W2_root),
                    jnp.tile(b2, 8).reshape(1, 128))
    part2 = _segsum(p2p.reshape(_N, _H), ei)
    gsum = _blockdiag8(jnp.ones((_H, _H), _f32))
    outp = _tc3(part2.reshape(_NC, _NACCP, 128), r2p, gsum)
    return outp.reshape(_N, _H)
